# Initial kernel scaffold; baseline (speedup 1.0000x reference)
import jax
import jax.numpy as jnp
from jax.experimental import pallas as pl


def kernel(x):
    raise NotImplementedError("write your pallas kernel here")


# speedup vs baseline: 2.7243x; 2.7243x over previous
import functools

import jax
import jax.numpy as jnp
from jax import lax
from jax.experimental import pallas as pl
from jax.experimental.pallas import tpu as pltpu
from jax.experimental.pallas import tpu_sc as plsc

_TOPK = 5
_RADIUS = 5
_ROWS = 64
_N = 32768
_L = 16
_C = 256
_NCHUNK = _N // _C
_NVPC = _C // _L
_BIG = 1 << 30

_mesh = plsc.VectorSubcoreMesh(
    core_axis_name="c", subcore_axis_name="s", num_cores=2, num_subcores=16
)


def _rescan_chunk(row_v, cmax_v, c):
    base = c * _C
    acc = row_v[pl.ds(base, _L)]
    for i in range(1, _NVPC):
        acc = jnp.maximum(acc, row_v[pl.ds(base + i * _L, _L)])
    cmax_v[pl.ds(c * _L, _L)] = acc


@functools.partial(
    pl.kernel,
    out_type=jax.ShapeDtypeStruct((_ROWS, _N), jnp.float32),
    mesh=_mesh,
    compiler_params=pltpu.CompilerParams(needs_layout_passes=False),
    scratch_types=[
        pltpu.VMEM((_N,), jnp.float32),
        pltpu.VMEM((_N,), jnp.float32),
        pltpu.VMEM((_NCHUNK * _L,), jnp.float32),
    ],
)
def _wta_sc(x_hbm, out_hbm, row_v, out_v, cmax_v):
    wid = lax.axis_index("s") * 2 + lax.axis_index("c")
    iota = lax.iota(jnp.int32, _L)
    zero_v = jnp.zeros((_L,), jnp.float32)
    one_v = jnp.ones((_L,), jnp.float32)

    def _zbody(i, carry):
        for u in range(16):
            out_v[pl.ds(i * 256 + u * _L, _L)] = zero_v
        return carry

    lax.fori_loop(0, _N // 256, _zbody, jnp.int32(0))

    for r in range(_ROWS // 32):
        row = wid * (_ROWS // 32) + r
        pltpu.sync_copy(x_hbm.at[row], row_v)

        def _cbody(c, gmin):
            base = c * _C
            acc = row_v[pl.ds(base, _L)]
            accmin = acc
            for i in range(1, _NVPC):
                v = row_v[pl.ds(base + i * _L, _L)]
                acc = jnp.maximum(acc, v)
                accmin = jnp.minimum(accmin, v)
            cmax_v[pl.ds(c * _L, _L)] = acc
            return jnp.minimum(gmin, accmin)

        gminv = lax.fori_loop(
            0, _NCHUNK, _cbody, jnp.full((_L,), jnp.inf, jnp.float32)
        )
        m = jnp.min(gminv)
        m_v = jnp.full((_L,), m, jnp.float32)

        picks = []
        for t in range(_TOPK):
            def _abody(i, acc):
                for u in range(8):
                    acc = jnp.maximum(acc, cmax_v[pl.ds((i * 8 + u) * _L, _L)])
                return acc

            maxacc = lax.fori_loop(
                0, _NCHUNK // 8, _abody, jnp.full((_L,), -jnp.inf, jnp.float32)
            )
            big_m = jnp.max(maxacc)

            def _bbody(i, acc):
                for u in range(8):
                    c = i * 8 + u
                    cm = cmax_v[pl.ds(c * _L, _L)]
                    acc = jnp.minimum(acc, jnp.where(cm == big_m, c, _BIG))
                return acc

            cidxv = lax.fori_loop(
                0, _NCHUNK // 8, _bbody, jnp.full((_L,), _BIG, jnp.int32)
            )
            cidx = jnp.min(cidxv)

            base = cidx * _C
            idxacc = jnp.full((_L,), _BIG, jnp.int32)
            for i in range(_NVPC):
                v = row_v[pl.ds(base + i * _L, _L)]
                idxacc = jnp.minimum(
                    idxacc, jnp.where(v == big_m, base + i * _L + iota, _BIG)
                )
            gidx = jnp.min(idxacc)
            picks.append(gidx)

            widx = gidx - _RADIUS + iota
            wmask = (iota < 2 * _RADIUS) & (gidx >= _RADIUS) & (widx < _N)
            widx_c = jnp.clip(widx, 0, _N - 1)
            plsc.store_scatter(row_v, [widx_c], m_v, mask=wmask)

            if t < _TOPK - 1:
                ws = jnp.maximum(gidx - _RADIUS, 0)
                we = jnp.minimum(gidx + _RADIUS, _N) - 1
                _rescan_chunk(row_v, cmax_v, ws // _C)
                _rescan_chunk(row_v, cmax_v, we // _C)

        pv = jnp.zeros((_L,), jnp.int32)
        vmask = iota == 0
        for t in range(_TOPK):
            pv = jnp.where(iota == t, picks[t], pv)
            if t > 0:
                dup = picks[t] == picks[0]
                for s in range(1, t):
                    dup = dup | (picks[t] == picks[s])
                vmask = vmask | ((iota == t) & jnp.logical_not(dup))

        plsc.store_scatter(out_v, [pv], one_v, mask=vmask)
        pltpu.sync_copy(out_v, out_hbm.at[row])
        plsc.store_scatter(out_v, [pv], zero_v, mask=vmask)


def kernel(x):
    return _wta_sc(x)
